# baseline (speedup 1.0000x reference)
import functools

import jax
import jax.numpy as jnp
from jax import lax
from jax.experimental import pallas as pl
from jax.experimental.pallas import tpu as pltpu
from jax.experimental.pallas import tpu_sc as plsc

EMBED_DIM = 64
NUM_TOKENS = 16384
SEQ = 50
TABLE_ROWS = 1000000
HALF = 524288

TC_CH = 2048
CHUNK = 512
TBL = CHUNK // 128


def _tc_detile(w_t):
    nblk = (HALF + TC_CH - 1) // TC_CH

    def body(x1_ref, x2_ref, out_ref):
        out_ref[:, 0:EMBED_DIM] = x1_ref[...].T
        out_ref[:, EMBED_DIM : 2 * EMBED_DIM] = x2_ref[...].T

    return pl.pallas_call(
        body,
        grid=(nblk,),
        in_specs=[
            pl.BlockSpec((EMBED_DIM, TC_CH), lambda i: (0, i)),
            pl.BlockSpec(
                (EMBED_DIM, TC_CH),
                lambda i: (
                    0,
                    jnp.minimum(i + HALF // TC_CH, TABLE_ROWS // TC_CH),
                ),
            ),
        ],
        out_specs=pl.BlockSpec((TC_CH, 128), lambda i: (i, 0)),
        out_shape=jax.ShapeDtypeStruct((HALF, 128), jnp.float32),
    )(w_t, w_t)


def _sc_gather(table, ids_t):
    info = plsc.get_sparse_core_info()
    nw = info.num_cores * info.num_subcores
    b_total = NUM_TOKENS * SEQ
    b_per_w = b_total // nw
    n_chunks = b_per_w // CHUNK
    n_pairs = n_chunks // 2
    mesh = plsc.VectorSubcoreMesh(core_axis_name="c", subcore_axis_name="s")

    pchunk = TBL * 1024
    psz = 8 * pchunk

    @functools.partial(
        pl.kernel,
        mesh=mesh,
        out_type=jax.ShapeDtypeStruct((SEQ * 8 * 128 * 8 * 128,), jnp.float32),
        compiler_params=pltpu.CompilerParams(
            use_tc_tiling_on_sc=False, needs_layout_passes=False
        ),
        scratch_types=[
            pltpu.VMEM((b_per_w,), jnp.int32),
            pltpu.VMEM((CHUNK, EMBED_DIM), jnp.float32),
            pltpu.VMEM((CHUNK, EMBED_DIM), jnp.float32),
            pltpu.VMEM((psz,), jnp.float32),
            pltpu.SemaphoreType.DMA,
            pltpu.SemaphoreType.DMA,
            pltpu.SemaphoreType.DMA,
        ],
    )
    def k(table_hbm, ids_hbm, out_hbm, idx_v, r0, r1, pv, sg0, sg1, sp):
        rows = (r0, r1)
        sg = (sg0, sg1)
        wid = lax.axis_index("s") * info.num_cores + lax.axis_index("c")
        w_base = wid * b_per_w
        pltpu.sync_copy(ids_hbm.at[pl.ds(w_base, b_per_w)], idx_v)
        iota = lax.iota(jnp.int32, 16)
        pidx = []
        for q in range(4):
            cvec = iota + 16 * q
            pidx.append((cvec >> 3) * pchunk + (cvec & 7) * 128)

        @plsc.parallel_loop(0, b_per_w, 16, unroll=8)
        def _(o):
            t = idx_v[pl.ds(o, 16)]
            idx_v[pl.ds(o, 16)] = jnp.where(t < HALF, 2 * t, 2 * t - (2 * HALF - 1))

        def gather_start(i, b):
            pltpu.async_copy(
                table_hbm.at[idx_v.at[pl.ds(i * CHUNK, CHUNK)]], rows[b], sg[b]
            )

        def gather_wait(b):
            pltpu.make_async_copy(
                table_hbm.at[idx_v.at[pl.ds(0, CHUNK)]], rows[b], sg[b]
            ).wait()

        def planes_start(i):
            f0 = w_base + i * CHUNK
            s = f0 // NUM_TOKENS
            tb0 = (f0 % NUM_TOKENS) // 128
            for c8 in range(8):
                pltpu.async_copy(
                    pv.at[pl.ds(c8 * pchunk, pchunk)],
                    out_hbm.at[pl.ds(((s * 8 + c8) * 128 + tb0) * 1024, pchunk)],
                    sp,
                )

        def planes_wait():
            pltpu.make_async_copy(out_hbm.at[pl.ds(0, psz)], pv, sp).wait()

        def transpose(b):
            rv = rows[b]

            @plsc.parallel_loop(0, CHUNK, 1, unroll=8)
            def _(j):
                off = ((j >> 7) << 10) + (j & 127)
                vecs = [rv[j, pl.ds(q * 16, 16)] for q in range(4)]
                for q in range(4):
                    plsc.store_scatter(pv, [pidx[q] + off], vecs[q])

        for b in range(2):
            gather_start(b, b)

        gather_wait(0)
        transpose(0)
        planes_start(0)
        gather_start(2, 0)
        gather_wait(1)
        planes_wait()
        transpose(1)
        planes_start(1)
        gather_start(3, 1)

        def pair(g, carry):
            for b in range(2):
                i = 2 * g + b
                gather_wait(b)
                planes_wait()
                transpose(b)
                planes_start(i)
                gather_start(i + 2, b)
            return carry

        lax.fori_loop(1, n_pairs - 1, pair, 0, unroll=False)

        for b in range(2):
            i = n_chunks - 2 + b
            gather_wait(b)
            planes_wait()
            transpose(b)
            planes_start(i)
        planes_wait()

    return k(table, ids_t)


def kernel(token_ids, weight):
    w_t = weight.T
    lin = _tc_detile(w_t)
    table = lin.reshape(2 * HALF, EMBED_DIM)
    ids_t = token_ids.T.reshape(NUM_TOKENS * SEQ).astype(jnp.int32)
    out_flat = _sc_gather(table, ids_t)
    out5 = out_flat.reshape(SEQ, 8, 128, 8, 128)
    return jnp.transpose(out5, (2, 4, 0, 1, 3)).reshape(NUM_TOKENS, SEQ, EMBED_DIM)


# speedup vs baseline: 2.2742x; 2.2742x over previous
import functools

import jax
import jax.numpy as jnp
from jax import lax
from jax.experimental import pallas as pl
from jax.experimental.pallas import tpu as pltpu
from jax.experimental.pallas import tpu_sc as plsc

EMBED_DIM = 64
NUM_TOKENS = 16384
SEQ = 50
TABLE_ROWS = 1000000
HALF = 524288

TC_CH = 2048
CHUNK = 256
TBL = CHUNK // 128


def _tc_detile(w_t):
    nblk = (HALF + TC_CH - 1) // TC_CH

    def body(x1_ref, x2_ref, out_ref):
        out_ref[:, 0:EMBED_DIM] = x1_ref[...].T
        out_ref[:, EMBED_DIM : 2 * EMBED_DIM] = x2_ref[...].T

    return pl.pallas_call(
        body,
        grid=(nblk,),
        in_specs=[
            pl.BlockSpec((EMBED_DIM, TC_CH), lambda i: (0, i)),
            pl.BlockSpec(
                (EMBED_DIM, TC_CH),
                lambda i: (
                    0,
                    jnp.minimum(i + HALF // TC_CH, TABLE_ROWS // TC_CH),
                ),
            ),
        ],
        out_specs=pl.BlockSpec((TC_CH, 128), lambda i: (i, 0)),
        out_shape=jax.ShapeDtypeStruct((HALF, 128), jnp.float32),
    )(w_t, w_t)


def _sc_gather(table, ids_t):
    info = plsc.get_sparse_core_info()
    nw = info.num_cores * info.num_subcores
    b_total = NUM_TOKENS * SEQ
    b_per_w = b_total // nw
    n_chunks = b_per_w // CHUNK
    n_pairs = n_chunks // 2
    mesh = plsc.VectorSubcoreMesh(core_axis_name="c", subcore_axis_name="s")

    pchunk = TBL * 1024
    psz = 8 * pchunk

    @functools.partial(
        pl.kernel,
        mesh=mesh,
        out_type=jax.ShapeDtypeStruct((SEQ * 8 * 128 * 8 * 128,), jnp.float32),
        compiler_params=pltpu.CompilerParams(
            use_tc_tiling_on_sc=False, needs_layout_passes=False
        ),
        scratch_types=[
            pltpu.VMEM((b_per_w,), jnp.int32),
            pltpu.VMEM((CHUNK, EMBED_DIM), jnp.float32),
            pltpu.VMEM((CHUNK, EMBED_DIM), jnp.float32),
            pltpu.VMEM((CHUNK, EMBED_DIM + 1), jnp.float32),
            pltpu.VMEM((psz,), jnp.float32),
            pltpu.SemaphoreType.DMA,
            pltpu.SemaphoreType.DMA,
            pltpu.SemaphoreType.DMA,
        ],
    )
    def k(table_hbm, ids_hbm, out_hbm, idx_v, r0, r1, rp, pv, sg0, sg1, sp):
        rows = (r0, r1)
        sg = (sg0, sg1)
        wid = lax.axis_index("s") * info.num_cores + lax.axis_index("c")
        w_base = wid * b_per_w
        pltpu.sync_copy(ids_hbm.at[pl.ds(w_base, b_per_w)], idx_v)
        iota = lax.iota(jnp.int32, 16)

        @plsc.parallel_loop(0, b_per_w, 16, unroll=8)
        def _(o):
            t = idx_v[pl.ds(o, 16)]
            idx_v[pl.ds(o, 16)] = jnp.where(t < HALF, 2 * t, 2 * t - (2 * HALF - 1))

        def gather_start(i, b):
            pltpu.async_copy(
                table_hbm.at[idx_v.at[pl.ds(i * CHUNK, CHUNK)]],
                rows[b],
                sg[b],
            )

        def gather_wait(b):
            pltpu.make_async_copy(
                table_hbm.at[idx_v.at[pl.ds(0, CHUNK)]], rows[b], sg[b]
            ).wait()

        def planes_start(i):
            f0 = w_base + i * CHUNK
            s = f0 // NUM_TOKENS
            tb0 = (f0 % NUM_TOKENS) // 128
            for c8 in range(8):
                pltpu.async_copy(
                    pv.at[pl.ds(c8 * pchunk, pchunk)],
                    out_hbm.at[pl.ds(((s * 8 + c8) * 128 + tb0) * 1024, pchunk)],
                    sp,
                )

        def planes_wait():
            pltpu.make_async_copy(out_hbm.at[pl.ds(0, psz)], pv, sp).wait()

        def transpose(b):
            rv = rows[b]

            @plsc.parallel_loop(0, CHUNK, 1, unroll=8)
            def _(j):
                for q in range(4):
                    rp[j, pl.ds(q * 16, 16)] = rv[j, pl.ds(q * 16, 16)]

            @plsc.parallel_loop(0, EMBED_DIM, 1, unroll=2)
            def _(c):
                colbase = (c >> 3) * pchunk + (c & 7) * 128
                csplat = jnp.zeros((16,), jnp.int32) + c
                for g in range(CHUNK // 16):
                    ridx = iota + 16 * g
                    vec = plsc.load_gather(rp, [ridx, csplat])
                    pos = colbase + ((g >> 3) << 10) + ((g & 7) * 16)
                    pv[pl.ds(pos, 16)] = vec

        for b in range(2):
            gather_start(b, b)

        gather_wait(0)
        transpose(0)
        planes_start(0)
        gather_start(2, 0)
        gather_wait(1)
        planes_wait()
        transpose(1)
        planes_start(1)
        gather_start(3, 1)

        def pair(g, carry):
            for b in range(2):
                i = 2 * g + b
                gather_wait(b)
                planes_wait()
                transpose(b)
                planes_start(i)
                gather_start(i + 2, b)
            return carry

        lax.fori_loop(1, n_pairs - 1, pair, 0, unroll=False)

        for b in range(2):
            i = n_chunks - 2 + b
            gather_wait(b)
            planes_wait()
            transpose(b)
            planes_start(i)
        planes_wait()

    return k(table, ids_t)


def kernel(token_ids, weight):
    w_t = weight.T
    lin = _tc_detile(w_t)
    table = lin.reshape(2 * HALF, EMBED_DIM)
    ids_t = token_ids.T.reshape(NUM_TOKENS * SEQ).astype(jnp.int32)
    out_flat = _sc_gather(table, ids_t)
    out5 = out_flat.reshape(SEQ, 8, 128, 8, 128)
    return jnp.transpose(out5, (2, 4, 0, 1, 3)).reshape(NUM_TOKENS, SEQ, EMBED_DIM)


# speedup vs baseline: 2.6099x; 1.1476x over previous
import functools

import jax
import jax.numpy as jnp
from jax import lax
from jax.experimental import pallas as pl
from jax.experimental.pallas import tpu as pltpu
from jax.experimental.pallas import tpu_sc as plsc

EMBED_DIM = 64
NUM_TOKENS = 16384
SEQ = 50
TABLE_ROWS = 1000000
HALF = 524288

TC_CH = 4096
CHUNK = 256
TBL = CHUNK // 128


def _tc_detile(w_t):
    nblk = (HALF + TC_CH - 1) // TC_CH

    def body(x1_ref, x2_ref, out_ref):
        ident = jnp.eye(EMBED_DIM, dtype=jnp.float32)
        dims = (((0,), (0,)), ((), ()))
        out_ref[:, 0:EMBED_DIM] = jax.lax.dot_general(
            x1_ref[...], ident, dims, preferred_element_type=jnp.float32
        )
        out_ref[:, EMBED_DIM : 2 * EMBED_DIM] = jax.lax.dot_general(
            x2_ref[...], ident, dims, preferred_element_type=jnp.float32
        )

    return pl.pallas_call(
        body,
        grid=(nblk,),
        in_specs=[
            pl.BlockSpec((EMBED_DIM, TC_CH), lambda i: (0, i)),
            pl.BlockSpec(
                (EMBED_DIM, TC_CH),
                lambda i: (
                    0,
                    jnp.minimum(i + HALF // TC_CH, TABLE_ROWS // TC_CH),
                ),
            ),
        ],
        out_specs=pl.BlockSpec((TC_CH, 128), lambda i: (i, 0)),
        out_shape=jax.ShapeDtypeStruct((HALF, 128), jnp.float32),
    )(w_t, w_t)


def _sc_gather(table, ids_t):
    info = plsc.get_sparse_core_info()
    nw = info.num_cores * info.num_subcores
    b_total = NUM_TOKENS * SEQ
    b_per_w = b_total // nw
    n_chunks = b_per_w // CHUNK
    n_pairs = n_chunks // 2
    mesh = plsc.VectorSubcoreMesh(core_axis_name="c", subcore_axis_name="s")

    pchunk = TBL * 1024
    psz = 8 * pchunk

    @functools.partial(
        pl.kernel,
        mesh=mesh,
        out_type=jax.ShapeDtypeStruct((SEQ * 8 * 128 * 8 * 128,), jnp.float32),
        compiler_params=pltpu.CompilerParams(
            use_tc_tiling_on_sc=False, needs_layout_passes=False
        ),
        scratch_types=[
            pltpu.VMEM((b_per_w,), jnp.int32),
            pltpu.VMEM((CHUNK, EMBED_DIM), jnp.float32),
            pltpu.VMEM((CHUNK, EMBED_DIM), jnp.float32),
            pltpu.VMEM((CHUNK, EMBED_DIM + 1), jnp.float32),
            pltpu.VMEM((psz,), jnp.float32),
            pltpu.SemaphoreType.DMA,
            pltpu.SemaphoreType.DMA,
            pltpu.SemaphoreType.DMA,
        ],
    )
    def k(table_hbm, ids_hbm, out_hbm, idx_v, r0, r1, rp, pv, sg0, sg1, sp):
        rows = (r0, r1)
        sg = (sg0, sg1)
        wid = lax.axis_index("s") * info.num_cores + lax.axis_index("c")
        w_base = wid * b_per_w
        pltpu.sync_copy(ids_hbm.at[pl.ds(w_base, b_per_w)], idx_v)
        iota = lax.iota(jnp.int32, 16)

        @plsc.parallel_loop(0, b_per_w, 16, unroll=8)
        def _(o):
            t = idx_v[pl.ds(o, 16)]
            idx_v[pl.ds(o, 16)] = jnp.where(t < HALF, 2 * t, 2 * t - (2 * HALF - 1))

        def gather_start(i, b):
            pltpu.async_copy(
                table_hbm.at[idx_v.at[pl.ds(i * CHUNK, CHUNK)]],
                rows[b],
                sg[b],
            )

        def gather_wait(b):
            pltpu.make_async_copy(
                table_hbm.at[idx_v.at[pl.ds(0, CHUNK)]], rows[b], sg[b]
            ).wait()

        def planes_start(i):
            f0 = w_base + i * CHUNK
            s = f0 // NUM_TOKENS
            tb0 = (f0 % NUM_TOKENS) // 128
            for c8 in range(8):
                pltpu.async_copy(
                    pv.at[pl.ds(c8 * pchunk, pchunk)],
                    out_hbm.at[pl.ds(((s * 8 + c8) * 128 + tb0) * 1024, pchunk)],
                    sp,
                )

        def planes_wait():
            pltpu.make_async_copy(out_hbm.at[pl.ds(0, psz)], pv, sp).wait()

        def transpose(b):
            rv = rows[b]

            @plsc.parallel_loop(0, CHUNK, 1, unroll=8)
            def _(j):
                for q in range(4):
                    rp[j, pl.ds(q * 16, 16)] = rv[j, pl.ds(q * 16, 16)]

            @plsc.parallel_loop(0, EMBED_DIM, 1, unroll=2)
            def _(c):
                colbase = (c >> 3) * pchunk + (c & 7) * 128
                csplat = jnp.zeros((16,), jnp.int32) + c
                for g in range(CHUNK // 16):
                    ridx = iota + 16 * g
                    vec = plsc.load_gather(rp, [ridx, csplat])
                    pos = colbase + ((g >> 3) << 10) + ((g & 7) * 16)
                    pv[pl.ds(pos, 16)] = vec

        for b in range(2):
            gather_start(b, b)

        gather_wait(0)
        transpose(0)
        planes_start(0)
        gather_start(2, 0)
        gather_wait(1)
        planes_wait()
        transpose(1)
        planes_start(1)
        gather_start(3, 1)

        def pair(g, carry):
            for b in range(2):
                i = 2 * g + b
                gather_wait(b)
                planes_wait()
                transpose(b)
                planes_start(i)
                gather_start(i + 2, b)
            return carry

        lax.fori_loop(1, n_pairs - 1, pair, 0, unroll=False)

        for b in range(2):
            i = n_chunks - 2 + b
            gather_wait(b)
            planes_wait()
            transpose(b)
            planes_start(i)
        planes_wait()

    return k(table, ids_t)


def kernel(token_ids, weight):
    w_t = weight.T
    lin = _tc_detile(w_t)
    table = lin.reshape(2 * HALF, EMBED_DIM)
    ids_t = token_ids.T.reshape(NUM_TOKENS * SEQ).astype(jnp.int32)
    out_flat = _sc_gather(table, ids_t)
    out5 = out_flat.reshape(SEQ, 8, 128, 8, 128)
    return jnp.transpose(out5, (2, 4, 0, 1, 3)).reshape(NUM_TOKENS, SEQ, EMBED_DIM)
